# baseline (speedup 1.0000x reference)
import functools

import jax
import jax.numpy as jnp
from jax import lax
from jax.experimental import pallas as pl
from jax.experimental.pallas import tpu as pltpu
from jax.experimental.pallas import tpu_sc as plsc

_T = 32768
_D = 4096
_E = 64
_EPS = 1e-06
_SCALE = float(_D) ** 0.5
_BLK = 1024

_NC = 2
_NS = 16
_L = 16
_NW = _NC * _NS
_TPW = _T // _NW
_NCHUNK = _TPW // _L


def _probs_body(x_ref, wt_ref, probs_ref, probs_t_ref):
    x = x_ref[...]
    mean = jnp.mean(x * x, axis=1, keepdims=True)
    normed = x * lax.rsqrt(mean + _EPS)
    logits = jnp.dot(normed, wt_ref[...], preferred_element_type=jnp.float32)
    m = jnp.max(logits, axis=1, keepdims=True)
    ex = jnp.exp(logits - m)
    probs = ex / jnp.sum(ex, axis=1, keepdims=True)
    probs_ref[...] = probs
    probs_t_ref[...] = probs.T


def _tc_probs(x, wt):
    grid = (_T // _BLK,)
    return pl.pallas_call(
        _probs_body,
        grid=grid,
        in_specs=[
            pl.BlockSpec((_BLK, _D), lambda i: (i, 0)),
            pl.BlockSpec((_D, _E), lambda i: (0, 0)),
        ],
        out_specs=[
            pl.BlockSpec((_BLK, _E), lambda i: (i, 0)),
            pl.BlockSpec((_E, _BLK), lambda i: (0, i)),
        ],
        out_shape=[
            jax.ShapeDtypeStruct((_T, _E), jnp.float32),
            jax.ShapeDtypeStruct((_E, _T), jnp.float32),
        ],
    )(x, wt)


def _sc_topk_body(probs_t_hbm, pes_hbm, w1_hbm, w2_hbm, i1_hbm, i2_hbm,
                  pt_v, pes_v, w1_v, w2_v, i1_v, i2_v):
    wid = lax.axis_index("s") * _NC + lax.axis_index("c")
    base = wid * _TPW
    pltpu.sync_copy(probs_t_hbm.at[:, pl.ds(base, _TPW)], pt_v)
    pltpu.sync_copy(pes_hbm, pes_v)

    pes_rows = [pes_v[pl.ds(k * _L, _L)] for k in range(_E // _L)]

    def chunk(c, carry):
        off = c * _L
        m1 = jnp.full((_L,), -1.0, jnp.float32)
        m2 = jnp.full((_L,), -1.0, jnp.float32)
        i1 = jnp.zeros((_L,), jnp.int32)
        i2 = jnp.zeros((_L,), jnp.int32)
        s1 = jnp.zeros((_L,), jnp.float32)
        s2 = jnp.zeros((_L,), jnp.float32)
        for e in range(_E):
            v = pt_v[e, pl.ds(off, _L)]
            pe = jnp.full((_L,), pes_rows[e // _L][e % _L], jnp.float32)
            ev = jnp.full((_L,), e, jnp.int32)
            gt1 = v > m1
            gt2 = v > m2
            m2 = jnp.where(gt1, m1, jnp.where(gt2, v, m2))
            i2 = jnp.where(gt1, i1, jnp.where(gt2, ev, i2))
            s2 = jnp.where(gt1, s1, jnp.where(gt2, pe, s2))
            m1 = jnp.where(gt1, v, m1)
            i1 = jnp.where(gt1, ev, i1)
            s1 = jnp.where(gt1, pe, s1)
        denom = m1 + m2
        w1_v[pl.ds(off, _L)] = (m1 / denom) * s1
        w2_v[pl.ds(off, _L)] = (m2 / denom) * s2
        i1_v[pl.ds(off, _L)] = i1
        i2_v[pl.ds(off, _L)] = i2
        return carry

    lax.fori_loop(0, _NCHUNK, chunk, 0)
    pltpu.sync_copy(w1_v, w1_hbm.at[pl.ds(base, _TPW)])
    pltpu.sync_copy(w2_v, w2_hbm.at[pl.ds(base, _TPW)])
    pltpu.sync_copy(i1_v, i1_hbm.at[pl.ds(base, _TPW)])
    pltpu.sync_copy(i2_v, i2_hbm.at[pl.ds(base, _TPW)])


@functools.partial(
    pl.kernel,
    mesh=plsc.VectorSubcoreMesh(core_axis_name="c", subcore_axis_name="s"),
    out_type=[
        jax.ShapeDtypeStruct((_T,), jnp.float32),
        jax.ShapeDtypeStruct((_T,), jnp.float32),
        jax.ShapeDtypeStruct((_T,), jnp.int32),
        jax.ShapeDtypeStruct((_T,), jnp.int32),
    ],
    scratch_types=[
        pltpu.VMEM((_E, _TPW), jnp.float32),
        pltpu.VMEM((_E,), jnp.float32),
        pltpu.VMEM((_TPW,), jnp.float32),
        pltpu.VMEM((_TPW,), jnp.float32),
        pltpu.VMEM((_TPW,), jnp.int32),
        pltpu.VMEM((_TPW,), jnp.int32),
    ],
)
def _sc_topk(probs_t_hbm, pes_hbm, w1_hbm, w2_hbm, i1_hbm, i2_hbm,
             pt_v, pes_v, w1_v, w2_v, i1_v, i2_v):
    _sc_topk_body(probs_t_hbm, pes_hbm, w1_hbm, w2_hbm, i1_hbm, i2_hbm,
                  pt_v, pes_v, w1_v, w2_v, i1_v, i2_v)


@jax.jit
def kernel(x, norm_w, W, per_expert_scale):
    wt = (W * (norm_w * _SCALE)[None, :]).T
    probs, probs_t = _tc_probs(x, wt)
    w1, w2, i1, i2 = _sc_topk(probs_t, per_expert_scale)
    topw = jnp.stack([w1, w2], axis=1)
    topi = jnp.stack([i1, i2], axis=1)
    return (probs, topw, topi)
